# baseline (speedup 1.0000x reference)
import functools

import jax
import jax.numpy as jnp
from jax import lax
from jax.experimental import pallas as pl
from jax.experimental.pallas import tpu as pltpu
from jax.experimental.pallas import tpu_sc as plsc

NC, NS = 2, 16
NW = NC * NS
GB = 4
EP = 128


@functools.lru_cache(maxsize=None)
def _make_gather(b: int, l: int, emb: int):
    assert b % (NW * GB * 2) == 0
    lp = (l + 7) // 8 * 8
    b_per_w = b // NW
    n_groups = b_per_w // GB
    mesh = plsc.VectorSubcoreMesh(
        core_axis_name="c", subcore_axis_name="s",
        num_cores=NC, num_subcores=NS,
    )

    @functools.partial(
        pl.kernel,
        out_type=jax.ShapeDtypeStruct((b, lp, EP), jnp.float32),
        mesh=mesh,
        scratch_types=[
            pltpu.VMEM((b_per_w, l), jnp.int32),
            pltpu.VMEM((GB, l, EP), jnp.float32),
            pltpu.VMEM((GB, l, EP), jnp.float32),
            pltpu.SemaphoreType.DMA,
            pltpu.SemaphoreType.DMA,
        ],
        compiler_params=pltpu.CompilerParams(use_tc_tiling_on_sc=False),
    )
    def gather_kernel(idx_hbm, table_hbm, out_hbm, idx_v, buf0, buf1,
                      sem0, sem1):
        wid = lax.axis_index("s") * NC + lax.axis_index("c")
        base = wid * b_per_w
        pltpu.sync_copy(idx_hbm.at[pl.ds(base, b_per_w)], idx_v)

        def fire(g, buf, sem):
            for r in range(GB):
                pltpu.async_copy(
                    table_hbm.at[idx_v.at[g * GB + r]],
                    buf.at[r],
                    sem,
                )

        def drain_write(g, buf, sem):
            dst = out_hbm.at[pl.ds(base + g * GB, GB), pl.ds(0, l)]
            pltpu.make_async_copy(dst, buf, sem).wait()
            pltpu.sync_copy(buf, dst)

        fire(0, buf0, sem0)

        def body(i, carry):
            ga = 2 * i
            gb_ = 2 * i + 1
            fire(gb_, buf1, sem1)
            drain_write(ga, buf0, sem0)

            @pl.when(gb_ + 1 < n_groups)
            def _():
                fire(gb_ + 1, buf0, sem0)

            drain_write(gb_, buf1, sem1)
            return carry

        lax.fori_loop(0, n_groups // 2, body, 0)

    return gather_kernel


def kernel(x, table):
    b, l = x.shape
    v, emb = table.shape
    tablep = jnp.pad(table, ((0, 0), (0, EP - emb)))
    padded = _make_gather(b, l, emb)(x, tablep)
    return padded[:, :l, :emb]


# speedup vs baseline: 1.0863x; 1.0863x over previous
import functools

import jax
import jax.numpy as jnp
from jax import lax
from jax.experimental import pallas as pl
from jax.experimental.pallas import tpu as pltpu
from jax.experimental.pallas import tpu_sc as plsc

NC, NS = 2, 16
NW = NC * NS
GB = 4
LP, EP = 56, 128


@functools.lru_cache(maxsize=None)
def _make_gather(b: int, l: int, emb: int):
    assert b % (NW * GB * 2) == 0
    b_per_w = b // NW
    n_groups = b_per_w // GB
    mesh = plsc.VectorSubcoreMesh(
        core_axis_name="c", subcore_axis_name="s",
        num_cores=NC, num_subcores=NS,
    )

    @functools.partial(
        pl.kernel,
        out_type=jax.ShapeDtypeStruct((b, LP, EP), jnp.float32),
        mesh=mesh,
        scratch_types=[
            pltpu.VMEM((b_per_w, l), jnp.int32),
            pltpu.VMEM((GB, l, emb), jnp.float32),
            pltpu.VMEM((GB, l, emb), jnp.float32),
            pltpu.SemaphoreType.DMA,
            pltpu.SemaphoreType.DMA,
        ],
        compiler_params=pltpu.CompilerParams(use_tc_tiling_on_sc=False),
    )
    def gather_kernel(idx_hbm, table_hbm, out_hbm, idx_v, buf0, buf1,
                      sem0, sem1):
        wid = lax.axis_index("s") * NC + lax.axis_index("c")
        base = wid * b_per_w
        pltpu.sync_copy(idx_hbm.at[pl.ds(base, b_per_w)], idx_v)

        def fire(g, buf, sem):
            for r in range(GB):
                pltpu.async_copy(
                    table_hbm.at[idx_v.at[g * GB + r]],
                    buf.at[r],
                    sem,
                )

        def drain_write(g, buf, sem):
            dst = out_hbm.at[pl.ds(base + g * GB, GB), pl.ds(0, l), pl.ds(0, emb)]
            pltpu.make_async_copy(dst, buf, sem).wait()
            pltpu.sync_copy(buf, dst)

        fire(0, buf0, sem0)

        def body(i, carry):
            ga = 2 * i
            gb_ = 2 * i + 1
            fire(gb_, buf1, sem1)
            drain_write(ga, buf0, sem0)

            @pl.when(gb_ + 1 < n_groups)
            def _():
                fire(gb_ + 1, buf0, sem0)

            drain_write(gb_, buf1, sem1)
            return carry

        lax.fori_loop(0, n_groups // 2, body, 0)

    return gather_kernel


def kernel(x, table):
    b, l = x.shape
    emb = table.shape[1]
    padded = _make_gather(b, l, emb)(x, table)
    return padded[:, :l, :emb]


# speedup vs baseline: 1.0894x; 1.0028x over previous
import functools

import jax
import jax.numpy as jnp
from jax import lax
from jax.experimental import pallas as pl
from jax.experimental.pallas import tpu as pltpu
from jax.experimental.pallas import tpu_sc as plsc

NC, NS = 2, 16
NW = NC * NS
GB = 8
LP, EP = 56, 128


@functools.lru_cache(maxsize=None)
def _make_gather(b: int, l: int, emb: int):
    assert b % (NW * GB * 2) == 0
    b_per_w = b // NW
    n_groups = b_per_w // GB
    mesh = plsc.VectorSubcoreMesh(
        core_axis_name="c", subcore_axis_name="s",
        num_cores=NC, num_subcores=NS,
    )

    @functools.partial(
        pl.kernel,
        out_type=jax.ShapeDtypeStruct((b, LP, EP), jnp.float32),
        mesh=mesh,
        scratch_types=[
            pltpu.VMEM((b_per_w, l), jnp.int32),
            pltpu.VMEM((GB, l, emb), jnp.float32),
            pltpu.VMEM((GB, l, emb), jnp.float32),
            pltpu.SemaphoreType.DMA,
            pltpu.SemaphoreType.DMA,
        ],
        compiler_params=pltpu.CompilerParams(use_tc_tiling_on_sc=False),
    )
    def gather_kernel(idx_hbm, table_hbm, out_hbm, idx_v, buf0, buf1,
                      sem0, sem1):
        wid = lax.axis_index("s") * NC + lax.axis_index("c")
        base = wid * b_per_w
        pltpu.sync_copy(idx_hbm.at[pl.ds(base, b_per_w)], idx_v)

        def fire(g, buf, sem):
            for r in range(GB):
                pltpu.async_copy(
                    table_hbm.at[idx_v.at[g * GB + r]],
                    buf.at[r],
                    sem,
                )

        def drain_write(g, buf, sem):
            dst = out_hbm.at[pl.ds(base + g * GB, GB), pl.ds(0, l), pl.ds(0, emb)]
            pltpu.make_async_copy(dst, buf, sem).wait()
            pltpu.sync_copy(buf, dst)

        fire(0, buf0, sem0)

        def body(i, carry):
            ga = 2 * i
            gb_ = 2 * i + 1
            fire(gb_, buf1, sem1)
            drain_write(ga, buf0, sem0)

            @pl.when(gb_ + 1 < n_groups)
            def _():
                fire(gb_ + 1, buf0, sem0)

            drain_write(gb_, buf1, sem1)
            return carry

        lax.fori_loop(0, n_groups // 2, body, 0)

    return gather_kernel


def kernel(x, table):
    b, l = x.shape
    emb = table.shape[1]
    padded = _make_gather(b, l, emb)(x, table)
    return padded[:, :l, :emb]
